# baseline (speedup 1.0000x reference)
import functools

import jax
import jax.numpy as jnp
import numpy as np
from jax.experimental import pallas as pl
from jax.experimental.pallas import tpu as pltpu
from jax.sharding import Mesh, NamedSharding, PartitionSpec as P

_PAR = pltpu.CompilerParams(dimension_semantics=("parallel",))

N = 4096
C = 16
RA = 512
RB = 512


def _cast_body(ld_ref, lu_ref, x_ref, yd_ref, yu_ref, ldb_ref, lub_ref):
    ld = ld_ref[...]
    lu = lu_ref[...]
    x = x_ref[...]
    yd_ref[...] = jnp.dot(ld, x, preferred_element_type=jnp.float32)
    yu_ref[...] = jnp.dot(lu, x, preferred_element_type=jnp.float32)
    ldb_ref[...] = ld.astype(jnp.bfloat16)
    lub_ref[...] = lu.astype(jnp.bfloat16)


def _cast_stage(Ld, Lu, x):
    nl = Ld.shape[0]
    return pl.pallas_call(
        _cast_body,
        grid=(nl // RA,),
        in_specs=[
            pl.BlockSpec((RA, N), lambda i: (i, 0)),
            pl.BlockSpec((RA, N), lambda i: (i, 0)),
            pl.BlockSpec((N, C), lambda i: (0, 0)),
        ],
        out_specs=[
            pl.BlockSpec((RA, C), lambda i: (i, 0)),
            pl.BlockSpec((RA, C), lambda i: (i, 0)),
            pl.BlockSpec((RA, N), lambda i: (i, 0)),
            pl.BlockSpec((RA, N), lambda i: (i, 0)),
        ],
        out_shape=[
            jax.ShapeDtypeStruct((nl, C), jnp.float32),
            jax.ShapeDtypeStruct((nl, C), jnp.float32),
            jax.ShapeDtypeStruct((nl, N), jnp.bfloat16),
            jax.ShapeDtypeStruct((nl, N), jnp.bfloat16),
        ],
        compiler_params=_PAR,
    )(Ld, Lu, x)


def _pair_body(ld_ref, lu_ref, x_ref, yd_ref, yu_ref):
    x = x_ref[...].astype(jnp.bfloat16)
    yd_ref[...] = jnp.dot(ld_ref[...], x, preferred_element_type=jnp.float32)
    yu_ref[...] = jnp.dot(lu_ref[...], x, preferred_element_type=jnp.float32)


def _pair_stage(Ldb, Lub, h):
    nl = Ldb.shape[0]
    return pl.pallas_call(
        _pair_body,
        grid=(nl // RB,),
        in_specs=[
            pl.BlockSpec((RB, N), lambda i: (i, 0)),
            pl.BlockSpec((RB, N), lambda i: (i, 0)),
            pl.BlockSpec((N, C), lambda i: (0, 0)),
        ],
        out_specs=[
            pl.BlockSpec((RB, C), lambda i: (i, 0)),
            pl.BlockSpec((RB, C), lambda i: (i, 0)),
        ],
        out_shape=[
            jax.ShapeDtypeStruct((nl, C), jnp.float32),
            jax.ShapeDtypeStruct((nl, C), jnp.float32),
        ],
        compiler_params=_PAR,
    )(Ldb, Lub, h)


def _combine_body(ld_ref, lu_ref, xdf_ref, xuf_ref, xdl_ref, xul_ref,
                  x0l_ref, g_ref, b_ref, out_ref):
    xdf = xdf_ref[...].astype(jnp.bfloat16)
    xuf = xuf_ref[...].astype(jnp.bfloat16)
    xd2 = jnp.dot(ld_ref[...], xdf, preferred_element_type=jnp.float32)
    xu2 = jnp.dot(lu_ref[...], xuf, preferred_element_type=jnp.float32)
    acc = jnp.dot(x0l_ref[...], g_ref[0], preferred_element_type=jnp.float32)
    acc += jnp.dot(xdl_ref[...], g_ref[1], preferred_element_type=jnp.float32)
    acc += jnp.dot(xd2, g_ref[2], preferred_element_type=jnp.float32)
    acc += jnp.dot(xul_ref[...], g_ref[3], preferred_element_type=jnp.float32)
    acc += jnp.dot(xu2, g_ref[4], preferred_element_type=jnp.float32)
    out_ref[...] = acc + b_ref[...]


def _combine_stage(Ldb, Lub, xd_full, xu_full, xd_loc, xu_loc, x0_loc, G, b):
    nl = Ldb.shape[0]
    return pl.pallas_call(
        _combine_body,
        grid=(nl // RB,),
        in_specs=[
            pl.BlockSpec((RB, N), lambda i: (i, 0)),
            pl.BlockSpec((RB, N), lambda i: (i, 0)),
            pl.BlockSpec((N, C), lambda i: (0, 0)),
            pl.BlockSpec((N, C), lambda i: (0, 0)),
            pl.BlockSpec((RB, C), lambda i: (i, 0)),
            pl.BlockSpec((RB, C), lambda i: (i, 0)),
            pl.BlockSpec((RB, C), lambda i: (i, 0)),
            pl.BlockSpec((5, C, C), lambda i: (0, 0, 0)),
            pl.BlockSpec((1, C), lambda i: (0, 0)),
        ],
        out_specs=pl.BlockSpec((RB, C), lambda i: (i, 0)),
        out_shape=jax.ShapeDtypeStruct((nl, C), jnp.float32),
        compiler_params=_PAR,
    )(Ldb, Lub, xd_full, xu_full, xd_loc, xu_loc, x0_loc, G, b)


def _chain(Ld_loc, Lu_loc, x, G1, V2, zb, b2, sharded):
    nl = Ld_loc.shape[0]
    if sharded:
        idx = jax.lax.axis_index("d")
        gather = functools.partial(jax.lax.all_gather, axis_name="d",
                                   axis=0, tiled=True)
        x0_loc = jax.lax.dynamic_slice_in_dim(x, idx * nl, nl, axis=0)
    else:
        gather = lambda a: a
        x0_loc = x

    xd1_loc, xu1_loc, Ldb, Lub = _cast_stage(Ld_loc, Lu_loc, x)
    xd1 = gather(xd1_loc)
    xu1 = gather(xu1_loc)
    h_loc = _combine_stage(Ldb, Lub, xd1, xu1, xd1_loc, xu1_loc, x0_loc,
                           G1, zb)
    h = gather(h_loc)
    hd1_loc, hu1_loc = _pair_stage(Ldb, Lub, h)
    hd1 = gather(hd1_loc)
    hu1 = gather(hu1_loc)
    return _combine_stage(Ldb, Lub, hd1, hu1, hd1_loc, hu1_loc, h_loc,
                          V2, b2)


def kernel(x, laplacian_down, laplacian_up, W1, W2, W_lin, b_lin):
    G1 = jnp.transpose(W1, (2, 0, 1))
    V2 = jnp.einsum("iok,oj->kij", W2, W_lin)
    zb = jnp.zeros((1, C), jnp.float32)
    b2 = b_lin.reshape(1, C).astype(jnp.float32)

    devs = jax.devices()
    if len(devs) >= 2:
        mesh = Mesh(np.array(devs[:2]), ("d",))
        shard = NamedSharding(mesh, P("d", None))
        rep = NamedSharding(mesh, P(None, None))
        Ld = jax.device_put(laplacian_down, shard)
        Lu = jax.device_put(laplacian_up, shard)
        xr = jax.device_put(x, rep)
        G1r = jax.device_put(G1, NamedSharding(mesh, P(None, None, None)))
        V2r = jax.device_put(V2, NamedSharding(mesh, P(None, None, None)))
        zbr = jax.device_put(zb, rep)
        b2r = jax.device_put(b2, rep)
        f = jax.shard_map(
            functools.partial(_chain, sharded=True),
            mesh=mesh,
            in_specs=(P("d", None), P("d", None), P(None, None),
                      P(None, None, None), P(None, None, None),
                      P(None, None), P(None, None)),
            out_specs=P("d", None),
            check_vma=False,
        )
        return f(Ld, Lu, xr, G1r, V2r, zbr, b2r)
    return _chain(laplacian_down, laplacian_up, x, G1, V2, zb, b2,
                  sharded=False)


# speedup vs baseline: 4.4625x; 4.4625x over previous
import jax
import jax.numpy as jnp
from jax.experimental import pallas as pl
from jax.experimental.pallas import tpu as pltpu

N = 4096
C = 16
RA = 128
RB = 512

_F32 = jnp.float32
_BF16 = jnp.bfloat16


def _dot(a, b):
    return jnp.dot(a, b, preferred_element_type=_F32)


def _mega_body(x_ref, g1_ref, v2_ref, b2_ref, ld_hbm, lu_hbm,
               out_ref, lub_hbm,
               ldb_v, xd1_v, xu1_v, h_v, hd1_v, hu1_v):
    x = x_ref[...]

    def a_body(ld_ref, lu_ref, lub_ref):
        i = pl.program_id(0)
        rows = pl.ds(i * RA, RA)
        ld = ld_ref[...]
        lu = lu_ref[...]
        xd1_v[rows, :] = _dot(ld, x)
        xu1_v[rows, :] = _dot(lu, x)
        ldb_v[rows, :] = ld.astype(_BF16)
        lub_ref[...] = lu.astype(_BF16)

    pltpu.emit_pipeline(
        a_body,
        grid=(N // RA,),
        in_specs=[
            pl.BlockSpec((RA, N), lambda i: (i, 0)),
            pl.BlockSpec((RA, N), lambda i: (i, 0)),
        ],
        out_specs=[pl.BlockSpec((RA, N), lambda i: (i, 0))],
    )(ld_hbm, lu_hbm, lub_hbm)

    def b_body(lub_ref):
        i = pl.program_id(0)
        rows = pl.ds(i * RB, RB)
        xd2 = _dot(ldb_v[rows, :], xd1_v[...].astype(_BF16))
        xu2 = _dot(lub_ref[...], xu1_v[...].astype(_BF16))
        acc = _dot(x_ref[rows, :], g1_ref[0])
        acc += _dot(xd1_v[rows, :], g1_ref[1])
        acc += _dot(xd2, g1_ref[2])
        acc += _dot(xu1_v[rows, :], g1_ref[3])
        acc += _dot(xu2, g1_ref[4])
        h_v[rows, :] = acc

    def c_body(lub_ref):
        i = pl.program_id(0)
        rows = pl.ds(i * RB, RB)
        hb = h_v[...].astype(_BF16)
        hd1_v[rows, :] = _dot(ldb_v[rows, :], hb)
        hu1_v[rows, :] = _dot(lub_ref[...], hb)

    def d_body(lub_ref):
        i = pl.program_id(0)
        rows = pl.ds(i * RB, RB)
        hd2 = _dot(ldb_v[rows, :], hd1_v[...].astype(_BF16))
        hu2 = _dot(lub_ref[...], hu1_v[...].astype(_BF16))
        acc = _dot(h_v[rows, :], v2_ref[0])
        acc += _dot(hd1_v[rows, :], v2_ref[1])
        acc += _dot(hd2, v2_ref[2])
        acc += _dot(hu1_v[rows, :], v2_ref[3])
        acc += _dot(hu2, v2_ref[4])
        out_ref[rows, :] = acc + b2_ref[...]

    lub_spec = [pl.BlockSpec((RB, N), lambda i: (i, 0))]
    for body in (b_body, c_body, d_body):
        pltpu.emit_pipeline(body, grid=(N // RB,), in_specs=lub_spec)(lub_hbm)


def kernel(x, laplacian_down, laplacian_up, W1, W2, W_lin, b_lin):
    G1 = jnp.transpose(W1, (2, 0, 1))
    V2 = jnp.einsum("iok,oj->kij", W2, W_lin)
    b2 = b_lin.reshape(1, C).astype(_F32)

    out, _ = pl.pallas_call(
        _mega_body,
        in_specs=[
            pl.BlockSpec(memory_space=pltpu.MemorySpace.VMEM),
            pl.BlockSpec(memory_space=pltpu.MemorySpace.VMEM),
            pl.BlockSpec(memory_space=pltpu.MemorySpace.VMEM),
            pl.BlockSpec(memory_space=pltpu.MemorySpace.VMEM),
            pl.BlockSpec(memory_space=pltpu.MemorySpace.HBM),
            pl.BlockSpec(memory_space=pltpu.MemorySpace.HBM),
        ],
        out_specs=[
            pl.BlockSpec(memory_space=pltpu.MemorySpace.VMEM),
            pl.BlockSpec(memory_space=pltpu.MemorySpace.HBM),
        ],
        out_shape=[
            jax.ShapeDtypeStruct((N, C), _F32),
            jax.ShapeDtypeStruct((N, N), _BF16),
        ],
        compiler_params=pltpu.CompilerParams(
            vmem_limit_bytes=64 * 1024 * 1024),
        scratch_shapes=[
            pltpu.VMEM((N, N), _BF16),
            pltpu.VMEM((N, C), _F32),
            pltpu.VMEM((N, C), _F32),
            pltpu.VMEM((N, C), _F32),
            pltpu.VMEM((N, C), _F32),
            pltpu.VMEM((N, C), _F32),
        ],
    )(x, G1, V2, b2, laplacian_down, laplacian_up)
    return out
